# baseline (speedup 1.0000x reference)
import functools

import jax
import jax.numpy as jnp
from jax import lax
from jax.experimental import pallas as pl
from jax.experimental.pallas import tpu as pltpu
from jax.experimental.pallas import tpu_sc as plsc

D = 1024
KGRP = 4
N_TOKENS = 8192
NW = 32
CHUNK = 32


def _sc_gather(table, idx, n_rows):
    rows_per_w = n_rows // NW
    nchunk = rows_per_w // CHUNK
    npair = nchunk // 2
    mesh = plsc.VectorSubcoreMesh(core_axis_name="c", subcore_axis_name="s")

    @functools.partial(
        pl.kernel,
        mesh=mesh,
        out_type=jax.ShapeDtypeStruct((n_rows, D), jnp.float32),
        scratch_types=[
            pltpu.VMEM((rows_per_w,), jnp.int32),
            pltpu.VMEM((CHUNK, D), jnp.float32),
            pltpu.VMEM((CHUNK, D), jnp.float32),
            pltpu.SemaphoreType.DMA,
            pltpu.SemaphoreType.DMA,
            pltpu.SemaphoreType.DMA,
            pltpu.SemaphoreType.DMA,
        ],
    )
    def gather_kernel(table_hbm, idx_hbm, out_hbm, idx_v, rows0, rows1,
                      gs0, gs1, os0, os1):
        wid = lax.axis_index("s") * 2 + lax.axis_index("c")
        base = wid * rows_per_w
        pltpu.sync_copy(idx_hbm.at[pl.ds(base, rows_per_w)], idx_v)
        bufs = (rows0, rows1)
        gss = (gs0, gs1)
        oss = (os0, os1)

        def g_args(c, b):
            return (table_hbm.at[idx_v.at[pl.ds(c * CHUNK, CHUNK)]],
                    bufs[b], gss[b])

        def w_args(c, b):
            return (bufs[b], out_hbm.at[pl.ds(base + c * CHUNK, CHUNK)],
                    oss[b])

        pltpu.async_copy(*g_args(0, 0))
        pltpu.async_copy(*g_args(1, 1))

        def body(j, carry):
            e = 2 * j
            o = e + 1
            pltpu.make_async_copy(*g_args(e, 0)).wait()
            pltpu.async_copy(*w_args(e, 0))
            pltpu.make_async_copy(*g_args(o, 1)).wait()
            pltpu.async_copy(*w_args(o, 1))
            pltpu.make_async_copy(*w_args(e, 0)).wait()
            pltpu.async_copy(*g_args(e + 2, 0))
            pltpu.make_async_copy(*w_args(o, 1)).wait()
            pltpu.async_copy(*g_args(o + 2, 1))
            return carry

        lax.fori_loop(0, npair - 1, body, 0)

        e = nchunk - 2
        o = nchunk - 1
        pltpu.make_async_copy(*g_args(e, 0)).wait()
        pltpu.async_copy(*w_args(e, 0))
        pltpu.make_async_copy(*g_args(o, 1)).wait()
        pltpu.async_copy(*w_args(o, 1))
        pltpu.make_async_copy(*w_args(e, 0)).wait()
        pltpu.make_async_copy(*w_args(o, 1)).wait()

    return gather_kernel(table, idx)


_TM = 512


def _tc_matmul(planes, w, b2d, n_tok):
    def body(a_ref, w_ref, b_ref, o_ref):
        acc = b_ref[...].astype(jnp.float32)
        acc = jnp.broadcast_to(acc, (_TM, D))
        for k in range(KGRP):
            a = a_ref[k].astype(jnp.bfloat16)
            wk = w_ref[:, k * D:(k + 1) * D]
            acc = acc + lax.dot_general(
                a, wk, (((1,), (1,)), ((), ())),
                preferred_element_type=jnp.float32,
            )
        o_ref[...] = acc

    return pl.pallas_call(
        body,
        grid=(n_tok // _TM,),
        in_specs=[
            pl.BlockSpec((KGRP, _TM, D), lambda i: (0, i, 0)),
            pl.BlockSpec((D, KGRP * D), lambda i: (0, 0)),
            pl.BlockSpec((1, D), lambda i: (0, 0)),
        ],
        out_specs=pl.BlockSpec((_TM, D), lambda i: (i, 0)),
        out_shape=jax.ShapeDtypeStruct((n_tok, D), jnp.float32),
    )(planes, w, b2d)


def kernel(x, emb_table, proj_w, proj_b):
    B, L, K = x.shape
    w_bf = proj_w.astype(jnp.bfloat16)
    b2d = proj_b.reshape(1, D)
    idx = x.transpose(0, 2, 1).reshape(B, K * L).astype(jnp.int32)
    outs = []
    for b in range(B):
        flat = _sc_gather(emb_table, idx[b], L * KGRP)
        planes = flat.reshape(KGRP, L, D)
        outs.append(_tc_matmul(planes, w_bf, b2d, L))
    return jnp.stack(outs, axis=0)


# speedup vs baseline: 1.1445x; 1.1445x over previous
import functools

import jax
import jax.numpy as jnp
from jax import lax
from jax.experimental import pallas as pl
from jax.experimental.pallas import tpu as pltpu
from jax.experimental.pallas import tpu_sc as plsc

D = 1024
KGRP = 4
N_TOKENS = 8192
NW = 32
CHUNK = 16
NSLICE = 4


def _sc_gather(table, idx, n_rows):
    rows_per_w = n_rows // NW
    nchunk = rows_per_w // CHUNK
    npair = nchunk // 2
    mesh = plsc.VectorSubcoreMesh(core_axis_name="c", subcore_axis_name="s")

    @functools.partial(
        pl.kernel,
        mesh=mesh,
        out_type=jax.ShapeDtypeStruct((n_rows, D), jnp.float32),
        scratch_types=[
            pltpu.VMEM((rows_per_w,), jnp.int32),
            pltpu.VMEM((CHUNK, D), jnp.float32),
            pltpu.VMEM((CHUNK, D), jnp.float32),
            pltpu.SemaphoreType.DMA,
            pltpu.SemaphoreType.DMA,
            pltpu.SemaphoreType.DMA,
            pltpu.SemaphoreType.DMA,
        ],
    )
    def gather_kernel(table_hbm, idx_hbm, out_hbm, idx_v, rows0, rows1,
                      gs0, gs1, os0, os1):
        wid = lax.axis_index("s") * 2 + lax.axis_index("c")
        base = wid * rows_per_w
        pltpu.sync_copy(idx_hbm.at[pl.ds(base, rows_per_w)], idx_v)
        bufs = (rows0, rows1)
        gss = (gs0, gs1)
        oss = (os0, os1)

        def g_args(c, b):
            return (table_hbm.at[idx_v.at[pl.ds(c * CHUNK, CHUNK)]],
                    bufs[b], gss[b])

        def w_args(c, b):
            return (bufs[b], out_hbm.at[pl.ds(base + c * CHUNK, CHUNK)],
                    oss[b])

        pltpu.async_copy(*g_args(0, 0))
        pltpu.async_copy(*g_args(1, 1))

        def body(j, carry):
            e = 2 * j
            o = e + 1
            pltpu.make_async_copy(*g_args(e, 0)).wait()
            pltpu.async_copy(*w_args(e, 0))
            pltpu.make_async_copy(*g_args(o, 1)).wait()
            pltpu.async_copy(*w_args(o, 1))
            pltpu.make_async_copy(*w_args(e, 0)).wait()
            pltpu.async_copy(*g_args(e + 2, 0))
            pltpu.make_async_copy(*w_args(o, 1)).wait()
            pltpu.async_copy(*g_args(o + 2, 1))
            return carry

        lax.fori_loop(0, npair - 1, body, 0)

        e = nchunk - 2
        o = nchunk - 1
        pltpu.make_async_copy(*g_args(e, 0)).wait()
        pltpu.async_copy(*w_args(e, 0))
        pltpu.make_async_copy(*g_args(o, 1)).wait()
        pltpu.async_copy(*w_args(o, 1))
        pltpu.make_async_copy(*w_args(e, 0)).wait()
        pltpu.make_async_copy(*w_args(o, 1)).wait()

    return gather_kernel(table, idx)


_TM = 1024


def _tc_matmul(planes, w, b2d, n_tok, tok_off, buf):
    tile_off = tok_off // _TM

    def body(*refs):
        a_ref, w_ref, b_ref = refs[0], refs[1], refs[2]
        o_ref = refs[-1]
        acc = b_ref[...].astype(jnp.float32)
        acc = jnp.broadcast_to(acc, (_TM, D))
        for k in range(KGRP):
            a = a_ref[k].astype(jnp.bfloat16)
            wk = w_ref[:, k * D:(k + 1) * D]
            acc = acc + lax.dot_general(
                a, wk, (((1,), (1,)), ((), ())),
                preferred_element_type=jnp.float32,
            )
        o_ref[...] = acc

    in_specs = [
        pl.BlockSpec((KGRP, _TM, D), lambda i: (0, i, 0)),
        pl.BlockSpec((D, KGRP * D), lambda i: (0, 0)),
        pl.BlockSpec((1, D), lambda i: (0, 0)),
    ]
    args = [planes, w, b2d]
    kwargs = {}
    if buf is not None:
        in_specs.append(pl.BlockSpec(memory_space=pl.ANY))
        args.append(buf)
        kwargs["input_output_aliases"] = {3: 0}

    return pl.pallas_call(
        body,
        grid=(n_tok // _TM,),
        in_specs=in_specs,
        out_specs=pl.BlockSpec((_TM, D), lambda i: (tile_off + i, 0)),
        out_shape=jax.ShapeDtypeStruct((N_TOKENS, D), jnp.float32),
        **kwargs,
    )(*args)


def kernel(x, emb_table, proj_w, proj_b):
    B, L, K = x.shape
    w_bf = proj_w.astype(jnp.bfloat16)
    b2d = proj_b.reshape(1, D)
    n_tok = N_TOKENS // NSLICE
    idx = x.reshape(N_TOKENS, K).T.reshape(K, NSLICE, n_tok)
    idx = idx.transpose(1, 0, 2).reshape(NSLICE, K * n_tok).astype(jnp.int32)
    buf = None
    for s in range(NSLICE):
        flat = _sc_gather(emb_table, idx[s], n_tok * KGRP)
        planes = flat.reshape(KGRP, n_tok, D)
        buf = _tc_matmul(planes, w_bf, b2d, n_tok, s * n_tok, buf)
    return buf.reshape(B, L, D)
